# baseline (speedup 1.0000x reference)
import functools

import jax
import jax.numpy as jnp
from jax import lax
from jax.experimental import pallas as pl
from jax.experimental.pallas import tpu as pltpu, tpu_sc as plsc

D_MODEL = 1024
TOTAL = 4 * 8192

_info = plsc.get_sparse_core_info()
NUM_WORKERS = _info.num_cores * _info.num_subcores
B_PER_W = TOTAL // NUM_WORKERS
CHUNK = 32
NBUF = 3
N_CHUNKS = B_PER_W // CHUNK


def _gather_kernel(pe_hbm, idx_hbm, out_hbm, idx_v, rows0, rows1, rows2,
                   gsem0, gsem1, gsem2, ssem0, ssem1, ssem2):
    wid = lax.axis_index("s") * _info.num_cores + lax.axis_index("c")
    base = pl.multiple_of(wid * B_PER_W, B_PER_W)
    pltpu.sync_copy(idx_hbm.at[pl.ds(base, B_PER_W)], idx_v)

    bufs = (rows0, rows1, rows2)
    gsems = (gsem0, gsem1, gsem2)
    ssems = (ssem0, ssem1, ssem2)

    def gstart(c, b):
        off = pl.multiple_of(c * CHUNK, CHUNK)
        pltpu.make_async_copy(
            pe_hbm.at[idx_v.at[pl.ds(off, CHUNK)]], bufs[b], gsems[b]
        ).start()

    def gwait(b):
        pltpu.make_async_copy(
            pe_hbm.at[idx_v.at[pl.ds(0, CHUNK)]], bufs[b], gsems[b]
        ).wait()

    def sstart(c, b):
        off = pl.multiple_of(c * CHUNK, CHUNK)
        pltpu.make_async_copy(
            bufs[b], out_hbm.at[pl.ds(base + off, CHUNK)], ssems[b]
        ).start()

    def swait(b):
        pltpu.make_async_copy(
            bufs[b], out_hbm.at[pl.ds(base, CHUNK)], ssems[b]
        ).wait()


    for b in range(NBUF - 1):
        gstart(b, b)

    for c in range(NBUF):
        b = c % NBUF
        pb = (b - 1) % NBUF
        gwait(b)
        sstart(c, b)
        if c > 0:
            swait(pb)
        gstart(c + NBUF - 1, pb)

    n_groups = N_CHUNKS // NBUF

    def body(s, _):
        for b in range(NBUF):
            c = s * NBUF + b
            pb = (b - 1) % NBUF
            gwait(b)
            sstart(c, b)
            swait(pb)
            gstart(c + NBUF - 1, pb)
        return 0

    lax.fori_loop(1, n_groups - 1, body, 0)

    for c in range((n_groups - 1) * NBUF, N_CHUNKS):
        b = c % NBUF
        pb = (b - 1) % NBUF
        gwait(b)
        sstart(c, b)
        if c + NBUF - 1 < N_CHUNKS:
            swait(pb)
            gstart(c + NBUF - 1, pb)

    for b in range(NBUF):
        swait(b)


@jax.jit
def _gather(pe, idx_flat):
    mesh = plsc.VectorSubcoreMesh(core_axis_name="c", subcore_axis_name="s")
    run = functools.partial(
        pl.kernel,
        mesh=mesh,
        out_type=jax.ShapeDtypeStruct((TOTAL, D_MODEL), jnp.float32),
        scratch_types=(
            [pltpu.VMEM((B_PER_W,), jnp.int32)]
            + [pltpu.VMEM((CHUNK, D_MODEL), jnp.float32)] * NBUF
            + [pltpu.SemaphoreType.DMA] * (2 * NBUF)
        ),
    )(_gather_kernel)
    return run(pe, idx_flat)


def kernel(positions, pe):
    idx_flat = positions.reshape(-1)
    out = _gather(pe, idx_flat)
    return out.reshape(positions.shape + (D_MODEL,))


# speedup vs baseline: 1.0073x; 1.0073x over previous
import functools

import jax
import jax.numpy as jnp
from jax import lax
from jax.experimental import pallas as pl
from jax.experimental.pallas import tpu as pltpu, tpu_sc as plsc

D_MODEL = 1024
TOTAL = 4 * 8192

_info = plsc.get_sparse_core_info()
NUM_WORKERS = _info.num_cores * _info.num_subcores
B_PER_W = TOTAL // NUM_WORKERS
CHUNK = 16
NBUF = 4
N_CHUNKS = B_PER_W // CHUNK


def _gather_kernel(pe_hbm, idx_hbm, out_hbm, idx_v, rows0, rows1, rows2, rows3,
                   gsem0, gsem1, gsem2, gsem3, ssem0, ssem1, ssem2, ssem3):
    wid = lax.axis_index("s") * _info.num_cores + lax.axis_index("c")
    base = pl.multiple_of(wid * B_PER_W, B_PER_W)
    pltpu.sync_copy(idx_hbm.at[pl.ds(base, B_PER_W)], idx_v)

    bufs = (rows0, rows1, rows2, rows3)
    gsems = (gsem0, gsem1, gsem2, gsem3)
    ssems = (ssem0, ssem1, ssem2, ssem3)

    def gstart(c, b):
        off = pl.multiple_of(c * CHUNK, CHUNK)
        pltpu.make_async_copy(
            pe_hbm.at[idx_v.at[pl.ds(off, CHUNK)]], bufs[b], gsems[b]
        ).start()

    def gwait(b):
        pltpu.make_async_copy(
            pe_hbm.at[idx_v.at[pl.ds(0, CHUNK)]], bufs[b], gsems[b]
        ).wait()

    def sstart(c, b):
        off = pl.multiple_of(c * CHUNK, CHUNK)
        pltpu.make_async_copy(
            bufs[b], out_hbm.at[pl.ds(base + off, CHUNK)], ssems[b]
        ).start()

    def swait(b):
        pltpu.make_async_copy(
            bufs[b], out_hbm.at[pl.ds(base, CHUNK)], ssems[b]
        ).wait()


    for b in range(NBUF - 1):
        gstart(b, b)

    for c in range(NBUF):
        b = c % NBUF
        pb = (b - 1) % NBUF
        gwait(b)
        sstart(c, b)
        if c > 0:
            swait(pb)
        gstart(c + NBUF - 1, pb)

    n_groups = N_CHUNKS // NBUF

    def body(s, _):
        for b in range(NBUF):
            c = s * NBUF + b
            pb = (b - 1) % NBUF
            gwait(b)
            sstart(c, b)
            swait(pb)
            gstart(c + NBUF - 1, pb)
        return 0

    lax.fori_loop(1, n_groups - 1, body, 0)

    for c in range((n_groups - 1) * NBUF, N_CHUNKS):
        b = c % NBUF
        pb = (b - 1) % NBUF
        gwait(b)
        sstart(c, b)
        if c + NBUF - 1 < N_CHUNKS:
            swait(pb)
            gstart(c + NBUF - 1, pb)

    for b in range(NBUF):
        swait(b)


@jax.jit
def _gather(pe, idx_flat):
    mesh = plsc.VectorSubcoreMesh(core_axis_name="c", subcore_axis_name="s")
    run = functools.partial(
        pl.kernel,
        mesh=mesh,
        out_type=jax.ShapeDtypeStruct((TOTAL, D_MODEL), jnp.float32),
        scratch_types=(
            [pltpu.VMEM((B_PER_W,), jnp.int32)]
            + [pltpu.VMEM((CHUNK, D_MODEL), jnp.float32)] * NBUF
            + [pltpu.SemaphoreType.DMA] * (2 * NBUF)
        ),
    )(_gather_kernel)
    return run(pe, idx_flat)


def kernel(positions, pe):
    idx_flat = positions.reshape(-1)
    out = _gather(pe, idx_flat)
    return out.reshape(positions.shape + (D_MODEL,))


# speedup vs baseline: 1.0094x; 1.0020x over previous
import functools

import jax
import jax.numpy as jnp
from jax import lax
from jax.experimental import pallas as pl
from jax.experimental.pallas import tpu as pltpu, tpu_sc as plsc

D_MODEL = 1024
TOTAL = 4 * 8192

_info = plsc.get_sparse_core_info()
NUM_WORKERS = _info.num_cores * _info.num_subcores
B_PER_W = TOTAL // NUM_WORKERS
CHUNK = 16
NBUF = 6
N_CHUNKS = B_PER_W // CHUNK


def _gather_kernel(pe_hbm, idx_hbm, out_hbm, idx_v, rows0, rows1, rows2, rows3,
                   rows4, rows5, gsem0, gsem1, gsem2, gsem3, gsem4, gsem5,
                   ssem0, ssem1, ssem2, ssem3, ssem4, ssem5):
    wid = lax.axis_index("s") * _info.num_cores + lax.axis_index("c")
    base = pl.multiple_of(wid * B_PER_W, B_PER_W)
    pltpu.sync_copy(idx_hbm.at[pl.ds(base, B_PER_W)], idx_v)

    bufs = (rows0, rows1, rows2, rows3, rows4, rows5)
    gsems = (gsem0, gsem1, gsem2, gsem3, gsem4, gsem5)
    ssems = (ssem0, ssem1, ssem2, ssem3, ssem4, ssem5)

    def gstart(c, b):
        off = pl.multiple_of(c * CHUNK, CHUNK)
        pltpu.make_async_copy(
            pe_hbm.at[idx_v.at[pl.ds(off, CHUNK)]], bufs[b], gsems[b]
        ).start()

    def gwait(b):
        pltpu.make_async_copy(
            pe_hbm.at[idx_v.at[pl.ds(0, CHUNK)]], bufs[b], gsems[b]
        ).wait()

    def sstart(c, b):
        off = pl.multiple_of(c * CHUNK, CHUNK)
        pltpu.make_async_copy(
            bufs[b], out_hbm.at[pl.ds(base + off, CHUNK)], ssems[b]
        ).start()

    def swait(b):
        pltpu.make_async_copy(
            bufs[b], out_hbm.at[pl.ds(base, CHUNK)], ssems[b]
        ).wait()


    for b in range(NBUF - 1):
        gstart(b, b)

    for c in range(NBUF):
        b = c % NBUF
        pb = (b - 1) % NBUF
        gwait(b)
        sstart(c, b)
        if c > 0:
            swait(pb)
        gstart(c + NBUF - 1, pb)

    n_groups = N_CHUNKS // NBUF

    def body(s, _):
        for b in range(NBUF):
            c = s * NBUF + b
            pb = (b - 1) % NBUF
            gwait(b)
            sstart(c, b)
            swait(pb)
            gstart(c + NBUF - 1, pb)
        return 0

    lax.fori_loop(1, n_groups - 1, body, 0)

    for c in range((n_groups - 1) * NBUF, N_CHUNKS):
        b = c % NBUF
        pb = (b - 1) % NBUF
        gwait(b)
        sstart(c, b)
        if c + NBUF - 1 < N_CHUNKS:
            swait(pb)
            gstart(c + NBUF - 1, pb)

    for b in range(NBUF):
        swait(b)


@jax.jit
def _gather(pe, idx_flat):
    mesh = plsc.VectorSubcoreMesh(core_axis_name="c", subcore_axis_name="s")
    run = functools.partial(
        pl.kernel,
        mesh=mesh,
        out_type=jax.ShapeDtypeStruct((TOTAL, D_MODEL), jnp.float32),
        scratch_types=(
            [pltpu.VMEM((B_PER_W,), jnp.int32)]
            + [pltpu.VMEM((CHUNK, D_MODEL), jnp.float32)] * NBUF
            + [pltpu.SemaphoreType.DMA] * (2 * NBUF)
        ),
    )(_gather_kernel)
    return run(pe, idx_flat)


def kernel(positions, pe):
    idx_flat = positions.reshape(-1)
    out = _gather(pe, idx_flat)
    return out.reshape(positions.shape + (D_MODEL,))
